# baseline (speedup 1.0000x reference)
import functools

import jax
import jax.numpy as jnp
from jax import lax
from jax.experimental import pallas as pl
from jax.experimental.pallas import tpu as pltpu
from jax.experimental.pallas import tpu_sc as plsc

_N = 10000
_D = 128
_EMB = 768
_NG = 64
_NOUT = 4
_NC, _NS = 2, 16
_NW = _NC * _NS
_CH = 128
_CPT = 80
_NCH = _NW * _CPT
_EP = _NCH * _CH
_NP = 10240
_STRIPE = _NP // _NS
_PAD_ROW = _N
_RB = 1024
_NBLK = _NP // _RB

_mesh = plsc.VectorSubcoreMesh(core_axis_name="c", subcore_axis_name="s",
                               num_cores=_NC, num_subcores=_NS)


_HALF = _NP // _NC
_NPL = _HALF
_LSTRIPE = _NPL // _NS
_CPT2 = _NCH // _NS
_PIECES = ((0, _CH), (_CH, _CH), (2 * _CH, _LSTRIPE - 2 * _CH))


def _stripe_zero(zbuf_v, sh, base):
    for off, rows in _PIECES:
        pltpu.sync_copy(zbuf_v.at[pl.ds(0, rows)],
                        sh.at[pl.ds(base + off, rows)])


def _stripe_readout(sh, bounce, out_hbm, c, base):
    for off, rows in _PIECES:
        pltpu.sync_copy(sh.at[pl.ds(base + off, rows)],
                        bounce.at[pl.ds(0, rows)])
        pltpu.sync_copy(bounce.at[pl.ds(0, rows)],
                        out_hbm.at[c, pl.ds(base + off, rows)])


def _compact_edges(src_v, dst_v, c):
    lo = c * _HALF

    def step(j, cnt):
        for g in range(_CH // 16):
            sl = pl.ds(g * 16, 16)
            sv = src_v[j, sl]
            dv = dst_v[j, sl] - lo
            ok = (dv >= 0) & (dv < _HALF)
            oki = ok.astype(jnp.int32)
            csum = plsc.cumsum(oki)
            pos = (cnt + csum) - oki
            prow = pos >> 7
            pcol = pos & (_CH - 1)
            plsc.store_scatter(src_v, [prow, pcol], sv, mask=ok)
            plsc.store_scatter(dst_v, [prow, pcol], dv, mask=ok)
            cnt = cnt + csum[15]
        return cnt

    cnt = lax.fori_loop(0, _CPT2, step, jnp.int32(0))

    zero16 = jnp.zeros((16,), jnp.int32)
    for r_base in (cnt >> 7 << 7, jnp.minimum((cnt >> 7) + 1, _CPT2 - 1) << 7):
        for g in range(_CH // 16):
            lane = lax.iota(jnp.int32, 16) + g * 16
            p = r_base + lane
            tail = p >= cnt
            plsc.store_scatter(src_v, [p >> 7, p & (_CH - 1)],
                               lane + _PAD_ROW, mask=tail)
            plsc.store_scatter(dst_v, [p >> 7, p & (_CH - 1)], zero16,
                               mask=tail)

    return cnt


@functools.partial(
    pl.kernel,
    out_type=jax.ShapeDtypeStruct((_CH, _D), jnp.float32),
    mesh=_mesh,
    scratch_types=[pltpu.VMEM((_CH, _D), jnp.float32)],
)
def _sc_warm(in_hbm, out_hbm, buf_v):
    c = lax.axis_index("c")
    s = lax.axis_index("s")

    @pl.when((c == 0) & (s == 0))
    def _():
        pltpu.sync_copy(in_hbm, buf_v)
        pltpu.sync_copy(buf_v, out_hbm)


@functools.partial(
    pl.kernel,
    out_type=jax.ShapeDtypeStruct((_NC, _NPL, _D), jnp.float32),
    mesh=_mesh,
    compiler_params=pltpu.CompilerParams(needs_layout_passes=False),
    scratch_types=[
        pltpu.VMEM((_CPT2, _CH), jnp.int32),
        pltpu.VMEM((_CPT2, _CH), jnp.int32),
        pltpu.VMEM((_CH, _D), jnp.float32),
        pltpu.VMEM((_CH, _D), jnp.float32),
        pltpu.VMEM((_CH, _D), jnp.float32),
        pltpu.VMEM_SHARED((_NPL, _D), jnp.float32),
        pltpu.SemaphoreType.DMA,
        pltpu.SemaphoreType.DMA,
    ],
)
def _sc_scatter(src_hbm, dst_hbm, y_hbm, zeros_hbm, z_out_hbm, src_v, dst_v,
                rows0_v, rows1_v, zbuf_v, z_sh, gsem0, gsem1):
    c = lax.axis_index("c")
    s = lax.axis_index("s")
    pltpu.sync_copy(src_hbm.at[pl.ds(s * _CPT2, _CPT2)], src_v)
    pltpu.sync_copy(dst_hbm.at[pl.ds(s * _CPT2, _CPT2)], dst_v)
    pltpu.sync_copy(zeros_hbm, zbuf_v)
    base = s * _LSTRIPE
    _stripe_zero(zbuf_v, z_sh, base)
    cnt = _compact_edges(src_v, dst_v, c)
    nch = (cnt + _CH - 1) >> 7
    plsc.subcore_barrier()

    pltpu.async_copy(y_hbm.at[src_v.at[0]], rows0_v, gsem0).wait()

    def body(t, carry):
        j = 2 * t
        d1 = pltpu.async_copy(y_hbm.at[src_v.at[j + 1]], rows1_v, gsem1)
        pltpu.sync_copy(rows0_v, z_sh.at[dst_v.at[j]], add=True)
        d1.wait()
        jn = jnp.where(j + 2 >= _CPT2, 0, j + 2)
        d0 = pltpu.async_copy(y_hbm.at[src_v.at[jn]], rows0_v, gsem0)
        pltpu.sync_copy(rows1_v, z_sh.at[dst_v.at[j + 1]], add=True)
        d0.wait()
        return carry

    lax.fori_loop(0, (nch + 1) >> 1, body, 0)
    plsc.subcore_barrier()
    _stripe_readout(z_sh, zbuf_v, z_out_hbm, c, base)


def _tc_scale_body(x_ref, deg_ref, y_ref, dinv_ref):
    d = deg_ref[0, :, 0:1] + 1.0
    dinv = lax.rsqrt(d)
    y_ref[...] = x_ref[...] * dinv
    dinv_ref[...] = jnp.broadcast_to(dinv, dinv_ref.shape)


_tc_scale = pl.pallas_call(
    _tc_scale_body,
    grid=(_NBLK,),
    in_specs=[
        pl.BlockSpec((_RB, _D), lambda i: (i, 0)),
        pl.BlockSpec((1, _RB, _D),
                     lambda i: (i // (_HALF // _RB), i % (_HALF // _RB), 0)),
    ],
    out_specs=[
        pl.BlockSpec((_RB, _D), lambda i: (i, 0)),
        pl.BlockSpec((_RB, 8), lambda i: (i, 0)),
    ],
    out_shape=[
        jax.ShapeDtypeStruct((_NP, _D), jnp.float32),
        jax.ShapeDtypeStruct((_NP, 8), jnp.float32),
    ],
)


def _tc_dense_body(zp_ref, y_ref, dinv_ref, batch_ref, wg_ref, bg_ref, wv_ref,
                   bv_ref, wo_ref, bo_ref, w1_ref, b1_ref, w2_ref, b2_ref,
                   out_ref, acc_ref):
    i = pl.program_id(0)

    @pl.when(i == 0)
    def _():
        acc_ref[...] = jnp.zeros_like(acc_ref)

    agg = (zp_ref[0] + y_ref[...]) * dinv_ref[:, 0:1]
    h = jnp.dot(agg, wg_ref[...], preferred_element_type=jnp.float32)
    h = jnp.maximum(h + bg_ref[...], 0.0)
    ids = lax.broadcasted_iota(jnp.int32, (_RB, _NG), 1)
    p = (batch_ref[...] == ids).astype(jnp.float32)
    haug = jnp.concatenate([h, jnp.ones((_RB, _D), jnp.float32)], axis=1)
    acc_ref[...] += lax.dot_general(
        p, haug, (((0,), (0,)), ((), ())), preferred_element_type=jnp.float32)

    @pl.when(i == _NBLK - 1)
    def _():
        acc = acc_ref[...]
        cnt = jnp.maximum(acc[:, _EMB:_EMB + 1], 1.0)
        g = acc[:, :_EMB] / cnt
        v = lax.dot_general(g, wv_ref[...], (((1,), (1,)), ((), ())),
                            preferred_element_type=jnp.float32) + bv_ref[...]
        a = lax.dot_general(v, wo_ref[...], (((1,), (1,)), ((), ())),
                            preferred_element_type=jnp.float32) + bo_ref[...]
        t = jnp.dot(a, w1_ref[...], preferred_element_type=jnp.float32)
        t = jnp.maximum(t + b1_ref[...], 0.0)
        o = jnp.dot(t, w2_ref[...],
                    preferred_element_type=jnp.float32) + b2_ref[...]
        m = jnp.max(o, axis=1, keepdims=True)
        e = jnp.exp(o - m)
        out_ref[...] = (o - m) - jnp.log(jnp.sum(e, axis=1, keepdims=True))


_tc_dense = pl.pallas_call(
    _tc_dense_body,
    grid=(_NBLK,),
    in_specs=[
        pl.BlockSpec((1, _RB, _D), lambda i: (i // (_HALF // _RB), i % (_HALF // _RB), 0)),
        pl.BlockSpec((_RB, _D), lambda i: (i, 0)),
        pl.BlockSpec((_RB, 8), lambda i: (i, 0)),
        pl.BlockSpec((_RB, 1), lambda i: (i, 0)),
        pl.BlockSpec((_D, _EMB), lambda i: (0, 0)),
        pl.BlockSpec((1, _EMB), lambda i: (0, 0)),
        pl.BlockSpec((_EMB, _EMB), lambda i: (0, 0)),
        pl.BlockSpec((1, _EMB), lambda i: (0, 0)),
        pl.BlockSpec((_EMB, _EMB), lambda i: (0, 0)),
        pl.BlockSpec((1, _EMB), lambda i: (0, 0)),
        pl.BlockSpec((_EMB, _EMB), lambda i: (0, 0)),
        pl.BlockSpec((1, _EMB), lambda i: (0, 0)),
        pl.BlockSpec((_EMB, _NOUT), lambda i: (0, 0)),
        pl.BlockSpec((1, _NOUT), lambda i: (0, 0)),
    ],
    out_specs=pl.BlockSpec((_NG, _NOUT), lambda i: (0, 0)),
    out_shape=jax.ShapeDtypeStruct((_NG, _NOUT), jnp.float32),
    scratch_shapes=[pltpu.VMEM((_NG, _EMB + _D), jnp.float32)],
)


def kernel(x, edge_index, batch, W_gcn, b_gcn, w_in, b_in, w_out, b_out, W1,
           b1, W2, b2):
    src = edge_index[0]
    dst = edge_index[1]
    padlen = _EP - src.shape[0]
    pad = jnp.full((padlen,), _PAD_ROW, jnp.int32)
    src_c = jnp.concatenate([src, pad]).reshape(_NCH, _CH)
    dst_c = jnp.concatenate([dst, pad]).reshape(_NCH, _CH)
    x_pad = jnp.pad(x, ((0, _NP - _N), (0, 0)))
    batch_pad = jnp.pad(batch, (0, _NP - _N),
                        constant_values=_NG).reshape(_NP, 1)
    ones128 = jnp.pad(jnp.ones((_N, _D), jnp.float32),
                      ((0, _NP - _N), (0, 0)))
    zeros128 = _sc_warm(jnp.zeros((_CH, _D), jnp.float32))

    degp = _sc_scatter(src_c, dst_c, ones128, zeros128)
    y, dinv = _tc_scale(x_pad, degp)
    zp = _sc_scatter(src_c, dst_c, y, zeros128)
    wv = w_in[2 * _EMB:3 * _EMB]
    bv = b_in[2 * _EMB:3 * _EMB].reshape(1, _EMB)
    return _tc_dense(zp, y, dinv, batch_pad, W_gcn, b_gcn.reshape(1, _EMB),
                     wv, bv, w_out, b_out.reshape(1, _EMB), W1,
                     b1.reshape(1, _EMB), W2, b2.reshape(1, _NOUT))


# speedup vs baseline: 1.5770x; 1.5770x over previous
import functools

import jax
import jax.numpy as jnp
from jax import lax
from jax.experimental import pallas as pl
from jax.experimental.pallas import tpu as pltpu
from jax.experimental.pallas import tpu_sc as plsc

_N = 10000
_D = 128
_EMB = 768
_NG = 64
_NOUT = 4
_NC, _NS = 2, 16
_NW = _NC * _NS
_CH = 128
_CPT = 80
_NCH = _NW * _CPT
_EP = _NCH * _CH
_NP = 10240
_STRIPE = _NP // _NS
_PAD_ROW = _N
_RB = 1024
_NBLK = _NP // _RB

_mesh = plsc.VectorSubcoreMesh(core_axis_name="c", subcore_axis_name="s",
                               num_cores=_NC, num_subcores=_NS)


_HALF = _NP // _NC
_NPL = _HALF
_LSTRIPE = _NPL // _NS
_CPT2 = _NCH // _NS
_PIECES = ((0, _CH), (_CH, _CH), (2 * _CH, _LSTRIPE - 2 * _CH))


def _stripe_zero(zbuf_v, sh, base):
    for off, rows in _PIECES:
        pltpu.sync_copy(zbuf_v.at[pl.ds(0, rows)],
                        sh.at[pl.ds(base + off, rows)])


def _stripe_readout(sh, bounce, out_hbm, c, base):
    for off, rows in _PIECES:
        pltpu.sync_copy(sh.at[pl.ds(base + off, rows)],
                        bounce.at[pl.ds(0, rows)])
        pltpu.sync_copy(bounce.at[pl.ds(0, rows)],
                        out_hbm.at[c, pl.ds(base + off, rows)])


def _compact_edges(src_v, dst_v, c):
    lo = c * _HALF

    def step(j, cnt):
        for g in range(_CH // 16):
            sl = pl.ds(g * 16, 16)
            sv = src_v[j, sl]
            dv = dst_v[j, sl] - lo
            ok = (dv >= 0) & (dv < _HALF)
            oki = ok.astype(jnp.int32)
            csum = plsc.cumsum(oki)
            pos = (cnt + csum) - oki
            prow = pos >> 7
            pcol = pos & (_CH - 1)
            plsc.store_scatter(src_v, [prow, pcol], sv, mask=ok)
            plsc.store_scatter(dst_v, [prow, pcol], dv, mask=ok)
            cnt = cnt + csum[15]
        return cnt

    cnt = lax.fori_loop(0, _CPT2, step, jnp.int32(0))

    zero16 = jnp.zeros((16,), jnp.int32)
    for r_base in (cnt >> 7 << 7, jnp.minimum((cnt >> 7) + 1, _CPT2 - 1) << 7):
        for g in range(_CH // 16):
            lane = lax.iota(jnp.int32, 16) + g * 16
            p = r_base + lane
            tail = p >= cnt
            plsc.store_scatter(src_v, [p >> 7, p & (_CH - 1)],
                               lane + _PAD_ROW, mask=tail)
            plsc.store_scatter(dst_v, [p >> 7, p & (_CH - 1)], zero16,
                               mask=tail)

    return cnt


def _compact_dst(dst_v, c):
    lo = c * _HALF

    def step(j, cnt):
        for g in range(_CH // 16):
            sl = pl.ds(g * 16, 16)
            dv = dst_v[j, sl] - lo
            ok = (dv >= 0) & (dv < _HALF)
            oki = ok.astype(jnp.int32)
            csum = plsc.cumsum(oki)
            pos = (cnt + csum) - oki
            plsc.store_scatter(dst_v, [pos >> 7, pos & (_CH - 1)], dv,
                               mask=ok)
            cnt = cnt + csum[15]
        return cnt

    cnt = lax.fori_loop(0, _CPT2, step, jnp.int32(0))
    zero16 = jnp.zeros((16,), jnp.int32)
    for g in range(_CH // 16):
        lane = lax.iota(jnp.int32, 16) + g * 16
        p = (cnt >> 7 << 7) + lane
        plsc.store_scatter(dst_v, [p >> 7, p & (_CH - 1)], zero16,
                           mask=p >= cnt)
    return cnt


@functools.partial(
    pl.kernel,
    out_type=jax.ShapeDtypeStruct((_NC, _NPL, _D), jnp.float32),
    mesh=_mesh,
    compiler_params=pltpu.CompilerParams(needs_layout_passes=False),
    scratch_types=[
        pltpu.VMEM((_CPT2, _CH), jnp.int32),
        pltpu.VMEM((_CH, _D), jnp.float32),
        pltpu.VMEM((_CH, _D), jnp.float32),
        pltpu.VMEM((_CH, _D), jnp.float32),
        pltpu.VMEM_SHARED((_NPL, _D), jnp.float32),
        pltpu.SemaphoreType.DMA,
    ],
)
def _sc_degree(dst_hbm, ones_hbm, zeros_hbm, deg_hbm, dst_v, ones_v, tail_v,
               zbuf_v, deg_sh, sem):
    c = lax.axis_index("c")
    s = lax.axis_index("s")
    pltpu.sync_copy(dst_hbm.at[pl.ds(s * _CPT2, _CPT2)], dst_v)
    pltpu.sync_copy(ones_hbm, ones_v)
    pltpu.sync_copy(zeros_hbm, zbuf_v)
    base = s * _LSTRIPE
    _stripe_zero(zbuf_v, deg_sh, base)
    cnt = _compact_dst(dst_v, c)
    nfull = cnt >> 7
    rem = cnt & (_CH - 1)

    def fill(r, carry):
        for g in range(_D // 16):
            keep = (lax.iota(jnp.int32, 16) * 0 + r) < rem
            tail_v[r, pl.ds(g * 16, 16)] = jnp.where(keep, 1.0, 0.0)
        return carry

    lax.fori_loop(0, _CH, fill, 0)
    plsc.subcore_barrier()

    def body(j, carry):
        pltpu.sync_copy(ones_v, deg_sh.at[dst_v.at[j]], add=True)
        return carry

    lax.fori_loop(0, nfull, body, 0)
    pltpu.sync_copy(tail_v, deg_sh.at[dst_v.at[jnp.minimum(nfull,
                                                           _CPT2 - 1)]],
                    add=True)
    plsc.subcore_barrier()
    _stripe_readout(deg_sh, zbuf_v, deg_hbm, c, base)


@functools.partial(
    pl.kernel,
    out_type=jax.ShapeDtypeStruct((_NC, _NPL, _D), jnp.float32),
    mesh=_mesh,
    compiler_params=pltpu.CompilerParams(needs_layout_passes=False),
    scratch_types=[
        pltpu.VMEM((_CPT2, _CH), jnp.int32),
        pltpu.VMEM((_CPT2, _CH), jnp.int32),
        pltpu.VMEM((_CH, _D), jnp.float32),
        pltpu.VMEM((_CH, _D), jnp.float32),
        pltpu.VMEM((_CH, _D), jnp.float32),
        pltpu.VMEM_SHARED((_NPL, _D), jnp.float32),
        pltpu.SemaphoreType.DMA,
        pltpu.SemaphoreType.DMA,
    ],
)
def _sc_scatter(src_hbm, dst_hbm, y_hbm, zeros_hbm, z_out_hbm, src_v, dst_v,
                rows0_v, rows1_v, zbuf_v, z_sh, gsem0, gsem1):
    c = lax.axis_index("c")
    s = lax.axis_index("s")
    pltpu.sync_copy(src_hbm.at[pl.ds(s * _CPT2, _CPT2)], src_v)
    pltpu.sync_copy(dst_hbm.at[pl.ds(s * _CPT2, _CPT2)], dst_v)
    pltpu.sync_copy(zeros_hbm, zbuf_v)
    base = s * _LSTRIPE
    _stripe_zero(zbuf_v, z_sh, base)
    cnt = _compact_edges(src_v, dst_v, c)
    nch = (cnt + _CH - 1) >> 7
    plsc.subcore_barrier()

    pltpu.async_copy(y_hbm.at[src_v.at[0]], rows0_v, gsem0).wait()

    def body(t, carry):
        j = 2 * t
        d1 = pltpu.async_copy(y_hbm.at[src_v.at[j + 1]], rows1_v, gsem1)
        pltpu.sync_copy(rows0_v, z_sh.at[dst_v.at[j]], add=True)
        d1.wait()
        jn = jnp.where(j + 2 >= _CPT2, 0, j + 2)
        d0 = pltpu.async_copy(y_hbm.at[src_v.at[jn]], rows0_v, gsem0)
        pltpu.sync_copy(rows1_v, z_sh.at[dst_v.at[j + 1]], add=True)
        d0.wait()
        return carry

    lax.fori_loop(0, (nch + 1) >> 1, body, 0)
    plsc.subcore_barrier()
    _stripe_readout(z_sh, zbuf_v, z_out_hbm, c, base)


def _tc_scale_body(x_ref, deg_ref, y_ref, dinv_ref):
    d = deg_ref[0, :, 0:1] + 1.0
    dinv = lax.rsqrt(d)
    y_ref[...] = x_ref[...] * dinv
    dinv_ref[...] = jnp.broadcast_to(dinv, dinv_ref.shape)


_tc_scale = pl.pallas_call(
    _tc_scale_body,
    grid=(_NBLK,),
    in_specs=[
        pl.BlockSpec((_RB, _D), lambda i: (i, 0)),
        pl.BlockSpec((1, _RB, _D),
                     lambda i: (i // (_HALF // _RB), i % (_HALF // _RB), 0)),
    ],
    out_specs=[
        pl.BlockSpec((_RB, _D), lambda i: (i, 0)),
        pl.BlockSpec((_RB, 8), lambda i: (i, 0)),
    ],
    out_shape=[
        jax.ShapeDtypeStruct((_NP, _D), jnp.float32),
        jax.ShapeDtypeStruct((_NP, 8), jnp.float32),
    ],
)


def _tc_dense_body(zp_ref, y_ref, dinv_ref, batch_ref, wg_ref, bg_ref, wv_ref,
                   bv_ref, wo_ref, bo_ref, w1_ref, b1_ref, w2_ref, b2_ref,
                   out_ref, acc_ref):
    i = pl.program_id(0)

    @pl.when(i == 0)
    def _():
        acc_ref[...] = jnp.zeros_like(acc_ref)

    agg = (zp_ref[0] + y_ref[...]) * dinv_ref[:, 0:1]
    h = jnp.dot(agg, wg_ref[...], preferred_element_type=jnp.float32)
    h = jnp.maximum(h + bg_ref[...], 0.0)
    ids = lax.broadcasted_iota(jnp.int32, (_RB, _NG), 1)
    p = (batch_ref[...] == ids).astype(jnp.float32)
    haug = jnp.concatenate([h, jnp.ones((_RB, _D), jnp.float32)], axis=1)
    acc_ref[...] += lax.dot_general(
        p, haug, (((0,), (0,)), ((), ())), preferred_element_type=jnp.float32)

    @pl.when(i == _NBLK - 1)
    def _():
        acc = acc_ref[...]
        cnt = jnp.maximum(acc[:, _EMB:_EMB + 1], 1.0)
        g = acc[:, :_EMB] / cnt
        v = lax.dot_general(g, wv_ref[...], (((1,), (1,)), ((), ())),
                            preferred_element_type=jnp.float32) + bv_ref[...]
        a = lax.dot_general(v, wo_ref[...], (((1,), (1,)), ((), ())),
                            preferred_element_type=jnp.float32) + bo_ref[...]
        t = jnp.dot(a, w1_ref[...], preferred_element_type=jnp.float32)
        t = jnp.maximum(t + b1_ref[...], 0.0)
        o = jnp.dot(t, w2_ref[...],
                    preferred_element_type=jnp.float32) + b2_ref[...]
        m = jnp.max(o, axis=1, keepdims=True)
        e = jnp.exp(o - m)
        out_ref[...] = (o - m) - jnp.log(jnp.sum(e, axis=1, keepdims=True))


_tc_dense = pl.pallas_call(
    _tc_dense_body,
    grid=(_NBLK,),
    in_specs=[
        pl.BlockSpec((1, _RB, _D), lambda i: (i // (_HALF // _RB), i % (_HALF // _RB), 0)),
        pl.BlockSpec((_RB, _D), lambda i: (i, 0)),
        pl.BlockSpec((_RB, 8), lambda i: (i, 0)),
        pl.BlockSpec((_RB, 1), lambda i: (i, 0)),
        pl.BlockSpec((_D, _EMB), lambda i: (0, 0)),
        pl.BlockSpec((1, _EMB), lambda i: (0, 0)),
        pl.BlockSpec((_EMB, _EMB), lambda i: (0, 0)),
        pl.BlockSpec((1, _EMB), lambda i: (0, 0)),
        pl.BlockSpec((_EMB, _EMB), lambda i: (0, 0)),
        pl.BlockSpec((1, _EMB), lambda i: (0, 0)),
        pl.BlockSpec((_EMB, _EMB), lambda i: (0, 0)),
        pl.BlockSpec((1, _EMB), lambda i: (0, 0)),
        pl.BlockSpec((_EMB, _NOUT), lambda i: (0, 0)),
        pl.BlockSpec((1, _NOUT), lambda i: (0, 0)),
    ],
    out_specs=pl.BlockSpec((_NG, _NOUT), lambda i: (0, 0)),
    out_shape=jax.ShapeDtypeStruct((_NG, _NOUT), jnp.float32),
    scratch_shapes=[pltpu.VMEM((_NG, _EMB + _D), jnp.float32)],
)


def kernel(x, edge_index, batch, W_gcn, b_gcn, w_in, b_in, w_out, b_out, W1,
           b1, W2, b2):
    src = edge_index[0]
    dst = edge_index[1]
    padlen = _EP - src.shape[0]
    pad = jnp.full((padlen,), _PAD_ROW, jnp.int32)
    src_c = jnp.concatenate([src, pad]).reshape(_NCH, _CH)
    dst_c = jnp.concatenate([dst, pad]).reshape(_NCH, _CH)
    x_pad = jnp.pad(x, ((0, _NP - _N), (0, 0)))
    batch_pad = jnp.pad(batch, (0, _NP - _N),
                        constant_values=_NG).reshape(_NP, 1)
    ones128 = jnp.ones((_CH, _D), jnp.float32)
    zeros128 = jnp.zeros((_CH, _D), jnp.float32)

    degp = _sc_degree(dst_c, ones128, zeros128)
    y, dinv = _tc_scale(x_pad, degp)
    zp = _sc_scatter(src_c, dst_c, y, zeros128)
    wv = w_in[2 * _EMB:3 * _EMB]
    bv = b_in[2 * _EMB:3 * _EMB].reshape(1, _EMB)
    return _tc_dense(zp, y, dinv, batch_pad, W_gcn, b_gcn.reshape(1, _EMB),
                     wv, bv, w_out, b_out.reshape(1, _EMB), W1,
                     b1.reshape(1, _EMB), W2, b2.reshape(1, _NOUT))


# speedup vs baseline: 1.6361x; 1.0375x over previous
import functools

import jax
import jax.numpy as jnp
from jax import lax
from jax.experimental import pallas as pl
from jax.experimental.pallas import tpu as pltpu
from jax.experimental.pallas import tpu_sc as plsc

_N = 10000
_D = 128
_EMB = 768
_NG = 64
_NOUT = 4
_NC, _NS = 2, 16
_NW = _NC * _NS
_CH = 128
_CPT = 80
_NCH = _NW * _CPT
_EP = _NCH * _CH
_NP = 10240
_STRIPE = _NP // _NS
_PAD_ROW = _N
_RB = 1024
_NBLK = _NP // _RB

_mesh = plsc.VectorSubcoreMesh(core_axis_name="c", subcore_axis_name="s",
                               num_cores=_NC, num_subcores=_NS)


_HALF = _NP // _NC
_NPL = _HALF
_LSTRIPE = _NPL // _NS
_CPT2 = _NCH // _NS
_PIECES = ((0, _CH), (_CH, _CH), (2 * _CH, _LSTRIPE - 2 * _CH))


def _stripe_zero(zbuf_v, sh, base):
    for off, rows in _PIECES:
        pltpu.sync_copy(zbuf_v.at[pl.ds(0, rows)],
                        sh.at[pl.ds(base + off, rows)])


def _stripe_readout(sh, bounce, out_hbm, c, base):
    for off, rows in _PIECES:
        pltpu.sync_copy(sh.at[pl.ds(base + off, rows)],
                        bounce.at[pl.ds(0, rows)])
        pltpu.sync_copy(bounce.at[pl.ds(0, rows)],
                        out_hbm.at[c, pl.ds(base + off, rows)])


def _compact_edges(src_v, dst_v, c):
    lo = c * _HALF

    def step(j, cnt):
        for g in range(_CH // 16):
            sl = pl.ds(g * 16, 16)
            sv = src_v[j, sl]
            dv = dst_v[j, sl] - lo
            ok = (dv >= 0) & (dv < _HALF)
            oki = ok.astype(jnp.int32)
            csum = plsc.cumsum(oki)
            pos = (cnt + csum) - oki
            prow = pos >> 7
            pcol = pos & (_CH - 1)
            plsc.store_scatter(src_v, [prow, pcol], sv, mask=ok)
            plsc.store_scatter(dst_v, [prow, pcol], dv, mask=ok)
            cnt = cnt + csum[15]
        return cnt

    cnt = lax.fori_loop(0, _CPT2, step, jnp.int32(0))

    zero16 = jnp.zeros((16,), jnp.int32)
    r0 = cnt >> 7
    for r_base in tuple(jnp.minimum(r0 + d, _CPT2 - 1) << 7
                        for d in range(5)):
        for g in range(_CH // 16):
            lane = lax.iota(jnp.int32, 16) + g * 16
            p = r_base + lane
            tail = p >= cnt
            plsc.store_scatter(src_v, [p >> 7, p & (_CH - 1)],
                               lane + _PAD_ROW, mask=tail)
            plsc.store_scatter(dst_v, [p >> 7, p & (_CH - 1)], zero16,
                               mask=tail)

    return cnt


def _compact_dst(dst_v, c):
    lo = c * _HALF

    def step(j, cnt):
        for g in range(_CH // 16):
            sl = pl.ds(g * 16, 16)
            dv = dst_v[j, sl] - lo
            ok = (dv >= 0) & (dv < _HALF)
            oki = ok.astype(jnp.int32)
            csum = plsc.cumsum(oki)
            pos = (cnt + csum) - oki
            plsc.store_scatter(dst_v, [pos >> 7, pos & (_CH - 1)], dv,
                               mask=ok)
            cnt = cnt + csum[15]
        return cnt

    cnt = lax.fori_loop(0, _CPT2, step, jnp.int32(0))
    zero16 = jnp.zeros((16,), jnp.int32)
    for g in range(_CH // 16):
        lane = lax.iota(jnp.int32, 16) + g * 16
        p = (cnt >> 7 << 7) + lane
        plsc.store_scatter(dst_v, [p >> 7, p & (_CH - 1)], zero16,
                           mask=p >= cnt)
    return cnt


@functools.partial(
    pl.kernel,
    out_type=jax.ShapeDtypeStruct((_NC, _NPL, _D), jnp.float32),
    mesh=_mesh,
    compiler_params=pltpu.CompilerParams(needs_layout_passes=False),
    scratch_types=[
        pltpu.VMEM((_CPT2, _CH), jnp.int32),
        pltpu.VMEM((_CH, _D), jnp.float32),
        pltpu.VMEM((_CH, _D), jnp.float32),
        pltpu.VMEM((_CH, _D), jnp.float32),
        pltpu.VMEM_SHARED((_NPL, _D), jnp.float32),
        pltpu.SemaphoreType.DMA,
    ],
)
def _sc_degree(dst_hbm, ones_hbm, zeros_hbm, deg_hbm, dst_v, ones_v, tail_v,
               zbuf_v, deg_sh, sem):
    c = lax.axis_index("c")
    s = lax.axis_index("s")
    pltpu.sync_copy(dst_hbm.at[pl.ds(s * _CPT2, _CPT2)], dst_v)
    pltpu.sync_copy(ones_hbm, ones_v)
    pltpu.sync_copy(zeros_hbm, zbuf_v)
    base = s * _LSTRIPE
    _stripe_zero(zbuf_v, deg_sh, base)
    cnt = _compact_dst(dst_v, c)
    nfull = cnt >> 7
    rem = cnt & (_CH - 1)

    def fill(r, carry):
        for g in range(_D // 16):
            keep = (lax.iota(jnp.int32, 16) * 0 + r) < rem
            tail_v[r, pl.ds(g * 16, 16)] = jnp.where(keep, 1.0, 0.0)
        return carry

    lax.fori_loop(0, _CH, fill, 0)
    plsc.subcore_barrier()

    def body(j, carry):
        pltpu.sync_copy(ones_v, deg_sh.at[dst_v.at[j]], add=True)
        return carry

    lax.fori_loop(0, nfull, body, 0)
    pltpu.sync_copy(tail_v, deg_sh.at[dst_v.at[jnp.minimum(nfull,
                                                           _CPT2 - 1)]],
                    add=True)
    plsc.subcore_barrier()
    _stripe_readout(deg_sh, zbuf_v, deg_hbm, c, base)


@functools.partial(
    pl.kernel,
    out_type=jax.ShapeDtypeStruct((_NC, _NPL, _D), jnp.float32),
    mesh=_mesh,
    compiler_params=pltpu.CompilerParams(needs_layout_passes=False),
    scratch_types=[
        pltpu.VMEM((_CPT2, _CH), jnp.int32),
        pltpu.VMEM((_CPT2, _CH), jnp.int32),
        pltpu.VMEM((_CH, _D), jnp.float32),
        pltpu.VMEM((_CH, _D), jnp.float32),
        pltpu.VMEM((_CH, _D), jnp.float32),
        pltpu.VMEM_SHARED((_NPL, _D), jnp.float32),
        pltpu.SemaphoreType.DMA,
        pltpu.SemaphoreType.DMA,
        pltpu.SemaphoreType.DMA,
    ],
)
def _sc_scatter(src_hbm, dst_hbm, y_hbm, zeros_hbm, z_out_hbm, src_v, dst_v,
                rows0_v, rows1_v, rows2_v, z_sh, gsem0, gsem1, gsem2):
    c = lax.axis_index("c")
    s = lax.axis_index("s")
    pltpu.sync_copy(src_hbm.at[pl.ds(s * _CPT2, _CPT2)], src_v)
    pltpu.sync_copy(dst_hbm.at[pl.ds(s * _CPT2, _CPT2)], dst_v)
    pltpu.sync_copy(zeros_hbm, rows2_v)
    base = s * _LSTRIPE
    _stripe_zero(rows2_v, z_sh, base)
    cnt = _compact_edges(src_v, dst_v, c)
    nch = (cnt + _CH - 1) >> 7
    plsc.subcore_barrier()

    bufs = (rows0_v, rows1_v, rows2_v)
    sems = (gsem0, gsem1, gsem2)
    for k in range(3):
        pltpu.async_copy(y_hbm.at[src_v.at[k]], bufs[k], sems[k])

    def body(t, carry):
        j = 3 * t
        for k in range(3):
            pltpu.make_async_copy(y_hbm.at[pl.ds(0, _CH)], bufs[k],
                                  sems[k]).wait()
            pltpu.sync_copy(bufs[k], z_sh.at[dst_v.at[j + k]], add=True)
            jn = jnp.minimum(j + k + 3, _CPT2 - 1)
            pltpu.async_copy(y_hbm.at[src_v.at[jn]], bufs[k], sems[k])
        return carry

    lax.fori_loop(0, (nch + 2) // 3, body, 0)
    for k in range(3):
        pltpu.make_async_copy(y_hbm.at[pl.ds(0, _CH)], bufs[k],
                              sems[k]).wait()
    plsc.subcore_barrier()
    _stripe_readout(z_sh, rows2_v, z_out_hbm, c, base)


def _tc_scale_body(x_ref, deg_ref, y_ref, dinv_ref):
    d = deg_ref[0, :, 0:1] + 1.0
    dinv = lax.rsqrt(d)
    y_ref[...] = x_ref[...] * dinv
    dinv_ref[...] = jnp.broadcast_to(dinv, dinv_ref.shape)


_tc_scale = pl.pallas_call(
    _tc_scale_body,
    grid=(_NBLK,),
    in_specs=[
        pl.BlockSpec((_RB, _D), lambda i: (i, 0)),
        pl.BlockSpec((1, _RB, _D),
                     lambda i: (i // (_HALF // _RB), i % (_HALF // _RB), 0)),
    ],
    out_specs=[
        pl.BlockSpec((_RB, _D), lambda i: (i, 0)),
        pl.BlockSpec((_RB, 8), lambda i: (i, 0)),
    ],
    out_shape=[
        jax.ShapeDtypeStruct((_NP, _D), jnp.float32),
        jax.ShapeDtypeStruct((_NP, 8), jnp.float32),
    ],
)


def _tc_dense_body(zp_ref, y_ref, dinv_ref, batch_ref, wg_ref, bg_ref, wv_ref,
                   bv_ref, wo_ref, bo_ref, w1_ref, b1_ref, w2_ref, b2_ref,
                   out_ref, acc_ref):
    i = pl.program_id(0)

    @pl.when(i == 0)
    def _():
        acc_ref[...] = jnp.zeros_like(acc_ref)

    agg = (zp_ref[0] + y_ref[...]) * dinv_ref[:, 0:1]
    h = jnp.dot(agg, wg_ref[...], preferred_element_type=jnp.float32)
    h = jnp.maximum(h + bg_ref[...], 0.0)
    ids = lax.broadcasted_iota(jnp.int32, (_RB, _NG), 1)
    p = (batch_ref[...] == ids).astype(jnp.float32)
    haug = jnp.concatenate([h, jnp.ones((_RB, _D), jnp.float32)], axis=1)
    acc_ref[...] += lax.dot_general(
        p, haug, (((0,), (0,)), ((), ())), preferred_element_type=jnp.float32)

    @pl.when(i == _NBLK - 1)
    def _():
        acc = acc_ref[...]
        cnt = jnp.maximum(acc[:, _EMB:_EMB + 1], 1.0)
        g = acc[:, :_EMB] / cnt
        v = lax.dot_general(g, wv_ref[...], (((1,), (1,)), ((), ())),
                            preferred_element_type=jnp.float32) + bv_ref[...]
        a = lax.dot_general(v, wo_ref[...], (((1,), (1,)), ((), ())),
                            preferred_element_type=jnp.float32) + bo_ref[...]
        t = jnp.dot(a, w1_ref[...], preferred_element_type=jnp.float32)
        t = jnp.maximum(t + b1_ref[...], 0.0)
        o = jnp.dot(t, w2_ref[...],
                    preferred_element_type=jnp.float32) + b2_ref[...]
        m = jnp.max(o, axis=1, keepdims=True)
        e = jnp.exp(o - m)
        out_ref[...] = (o - m) - jnp.log(jnp.sum(e, axis=1, keepdims=True))


_tc_dense = pl.pallas_call(
    _tc_dense_body,
    grid=(_NBLK,),
    in_specs=[
        pl.BlockSpec((1, _RB, _D), lambda i: (i // (_HALF // _RB), i % (_HALF // _RB), 0)),
        pl.BlockSpec((_RB, _D), lambda i: (i, 0)),
        pl.BlockSpec((_RB, 8), lambda i: (i, 0)),
        pl.BlockSpec((_RB, 1), lambda i: (i, 0)),
        pl.BlockSpec((_D, _EMB), lambda i: (0, 0)),
        pl.BlockSpec((1, _EMB), lambda i: (0, 0)),
        pl.BlockSpec((_EMB, _EMB), lambda i: (0, 0)),
        pl.BlockSpec((1, _EMB), lambda i: (0, 0)),
        pl.BlockSpec((_EMB, _EMB), lambda i: (0, 0)),
        pl.BlockSpec((1, _EMB), lambda i: (0, 0)),
        pl.BlockSpec((_EMB, _EMB), lambda i: (0, 0)),
        pl.BlockSpec((1, _EMB), lambda i: (0, 0)),
        pl.BlockSpec((_EMB, _NOUT), lambda i: (0, 0)),
        pl.BlockSpec((1, _NOUT), lambda i: (0, 0)),
    ],
    out_specs=pl.BlockSpec((_NG, _NOUT), lambda i: (0, 0)),
    out_shape=jax.ShapeDtypeStruct((_NG, _NOUT), jnp.float32),
    scratch_shapes=[pltpu.VMEM((_NG, _EMB + _D), jnp.float32)],
)


def kernel(x, edge_index, batch, W_gcn, b_gcn, w_in, b_in, w_out, b_out, W1,
           b1, W2, b2):
    src = edge_index[0]
    dst = edge_index[1]
    padlen = _EP - src.shape[0]
    pad = jnp.full((padlen,), _PAD_ROW, jnp.int32)
    src_c = jnp.concatenate([src, pad]).reshape(_NCH, _CH)
    dst_c = jnp.concatenate([dst, pad]).reshape(_NCH, _CH)
    x_pad = jnp.pad(x, ((0, _NP - _N), (0, 0)))
    batch_pad = jnp.pad(batch, (0, _NP - _N),
                        constant_values=_NG).reshape(_NP, 1)
    ones128 = jnp.ones((_CH, _D), jnp.float32)
    zeros128 = jnp.zeros((_CH, _D), jnp.float32)

    degp = _sc_degree(dst_c, ones128, zeros128)
    y, dinv = _tc_scale(x_pad, degp)
    zp = _sc_scatter(src_c, dst_c, y, zeros128)
    wv = w_in[2 * _EMB:3 * _EMB]
    bv = b_in[2 * _EMB:3 * _EMB].reshape(1, _EMB)
    return _tc_dense(zp, y, dinv, batch_pad, W_gcn, b_gcn.reshape(1, _EMB),
                     wv, bv, w_out, b_out.reshape(1, _EMB), W1,
                     b1.reshape(1, _EMB), W2, b2.reshape(1, _NOUT))
